# baseline (speedup 1.0000x reference)
import jax
import jax.numpy as jnp
from jax.experimental import pallas as pl


def _norm_block(x_ref, o_ref):
    x = x_ref[...]
    ss = jnp.sum(x * x, axis=-1, keepdims=True)
    o_ref[...] = x * jax.lax.rsqrt(jnp.maximum(ss, 1e-24))


def kernel(prototypes):
    m, d = prototypes.shape
    bm = 512
    return pl.pallas_call(
        _norm_block,
        grid=(m // bm,),
        in_specs=[pl.BlockSpec((bm, d), lambda i: (i, 0))],
        out_specs=pl.BlockSpec((bm, d), lambda i: (i, 0)),
        out_shape=jax.ShapeDtypeStruct((m, d), prototypes.dtype),
    )(prototypes)


# speedup vs baseline: 1.3804x; 1.3804x over previous
import jax
import jax.numpy as jnp
from jax.experimental import pallas as pl


def _norm_block(x_ref, o_ref):
    x = x_ref[...]
    ss = jnp.sum(x * x, axis=-1, keepdims=True)
    o_ref[...] = x * jax.lax.rsqrt(jnp.maximum(ss, 1e-24))


def kernel(prototypes):
    m, d = prototypes.shape
    bm = 1024
    return pl.pallas_call(
        _norm_block,
        grid=(m // bm,),
        in_specs=[pl.BlockSpec((bm, d), lambda i: (i, 0))],
        out_specs=pl.BlockSpec((bm, d), lambda i: (i, 0)),
        out_shape=jax.ShapeDtypeStruct((m, d), prototypes.dtype),
    )(prototypes)


# speedup vs baseline: 1.7752x; 1.2860x over previous
import jax
import jax.numpy as jnp
from jax.experimental import pallas as pl


def _norm_block(x_ref, o_ref):
    x = x_ref[...]
    ss = jnp.sum(x * x, axis=-1, keepdims=True)
    o_ref[...] = x * jax.lax.rsqrt(jnp.maximum(ss, 1e-24))


def kernel(prototypes):
    m, d = prototypes.shape
    bm = 2048
    return pl.pallas_call(
        _norm_block,
        grid=(m // bm,),
        in_specs=[pl.BlockSpec((bm, d), lambda i: (i, 0))],
        out_specs=pl.BlockSpec((bm, d), lambda i: (i, 0)),
        out_shape=jax.ShapeDtypeStruct((m, d), prototypes.dtype),
    )(prototypes)


# speedup vs baseline: 2.1824x; 1.2294x over previous
import jax
import jax.numpy as jnp
from jax.experimental import pallas as pl


def _norm_block(x_ref, o_ref):
    x = x_ref[...]
    ss = jnp.sum(x * x, axis=-1, keepdims=True)
    o_ref[...] = x * jax.lax.rsqrt(jnp.maximum(ss, 1e-24))


def kernel(prototypes):
    m, d = prototypes.shape
    bm = 4096
    return pl.pallas_call(
        _norm_block,
        grid=(m // bm,),
        in_specs=[pl.BlockSpec((bm, d), lambda i: (i, 0))],
        out_specs=pl.BlockSpec((bm, d), lambda i: (i, 0)),
        out_shape=jax.ShapeDtypeStruct((m, d), prototypes.dtype),
    )(prototypes)
